# baseline (speedup 1.0000x reference)
import functools

import jax
import jax.numpy as jnp
from jax import lax
from jax.experimental import pallas as pl
from jax.experimental.pallas import tpu as pltpu
from jax.experimental.pallas import tpu_sc as plsc

N = 10000
D_IN = 128
DG = 16
E1 = 106667
E = 3 * E1
NC = 2
NS = 16
NW = NC * NS
C = 640
CHUNKS = 16
T = C * CHUNKS
E_PAD = NW * T
S1 = 109568
S2 = 219136
NP = 10240
ROWS_PER_TILE = NP // NS


def _bdot(a, b):
    return jnp.dot(a.astype(jnp.bfloat16), b.astype(jnp.bfloat16),
                   preferred_element_type=jnp.float32)


def _matmul_body(x_ref, wl_ref, wr_ref, xl_ref, xr_ref):
    xb = x_ref[...]
    xl_ref[...] = _bdot(xb, wl_ref[...])
    xr_ref[...] = _bdot(xb, wr_ref[...])


def _project(x, W_l, W_r):
    blk = 2000
    return pl.pallas_call(
        _matmul_body,
        grid=(N // blk,),
        in_specs=[
            pl.BlockSpec((blk, D_IN), lambda i: (i, 0)),
            pl.BlockSpec((D_IN, DG), lambda i: (0, 0)),
            pl.BlockSpec((D_IN, DG), lambda i: (0, 0)),
        ],
        out_specs=[
            pl.BlockSpec((blk, DG), lambda i: (i, 0)),
            pl.BlockSpec((blk, DG), lambda i: (i, 0)),
        ],
        out_shape=[
            jax.ShapeDtypeStruct((N, DG), jnp.float32),
            jax.ShapeDtypeStruct((N, DG), jnp.float32),
        ],
    )(x, W_l, W_r)


def _bf16_round(v):
    u = plsc.bitcast(v, jnp.uint32)
    u = ((u + jnp.uint32(0x7FFF) + ((u >> jnp.uint32(16)) & jnp.uint32(1)))
         & jnp.uint32(0xFFFF0000))
    return plsc.bitcast(u, jnp.float32)


_RPC = C // 128


def _edge_kernel(xl_hbm, xr_hbm, idx_hbm, att_hbm, lew_hbm,
                 num_out, den_out,
                 iall_src, iall_dst, XL0, XR0, w0, XL1, XR1, w1,
                 XL2, XR2, w2, attv, lewv,
                 num_sp, den_sp, isem, gsem0, gsem1, gsem2,
                 ssem0, ssem1, ssem2):
    cid = lax.axis_index("c")
    tid = lax.axis_index("s")
    wid = tid * NC + cid
    zero16 = jnp.zeros((DG,), jnp.float32)

    idx_row0 = wid * (T // 128)
    cpi = pltpu.async_copy(idx_hbm.at[pl.ds(idx_row0, T // 128)],
                           iall_src, isem)
    cpd = pltpu.async_copy(idx_hbm.at[pl.ds(E_PAD // 128 + idx_row0, T // 128)],
                           iall_dst, isem)

    def _z_rows(i):
        XL0[i] = zero16
    plsc.parallel_loop(0, ROWS_PER_TILE, 1, unroll=8)(_z_rows)

    def _z_w(i):
        w0[pl.ds(i * DG, DG)] = zero16
    plsc.parallel_loop(0, ROWS_PER_TILE // DG, 1, unroll=8)(_z_w)

    row0 = tid * ROWS_PER_TILE
    pltpu.sync_copy(XL0.at[pl.ds(0, ROWS_PER_TILE)],
                    num_sp.at[pl.ds(row0, ROWS_PER_TILE)])
    pltpu.sync_copy(w0.at[pl.ds(0, ROWS_PER_TILE)],
                    den_sp.at[pl.ds(row0, ROWS_PER_TILE)])

    pltpu.sync_copy(att_hbm, attv)
    pltpu.sync_copy(lew_hbm, lewv)
    att_v = _bf16_round(attv[...])
    lew_v = lewv[...]

    cpi.wait()
    cpd.wait()
    plsc.subcore_barrier()

    edge0 = wid * T
    lane15 = lax.iota(jnp.int32, DG) == DG - 1

    def _issue_gather(g, XLb, XRb, semb):
        for j in range(_RPC):
            pltpu.async_copy(xl_hbm.at[iall_src.at[g * _RPC + j]],
                             XLb.at[pl.ds(j * 128, 128)], semb)
            pltpu.async_copy(xr_hbm.at[iall_dst.at[g * _RPC + j]],
                             XRb.at[pl.ds(j * 128, 128)], semb)

    def _wait_gather(XLb, XRb, semb):
        for j in range(_RPC):
            pltpu.make_async_copy(xl_hbm.at[pl.ds(0, 128)],
                                  XLb.at[pl.ds(j * 128, 128)], semb).wait()
            pltpu.make_async_copy(xr_hbm.at[pl.ds(0, 128)],
                                  XRb.at[pl.ds(j * 128, 128)], semb).wait()

    def _drain_scatter(XLb, wb, ssemb):
        for j in range(_RPC):
            pltpu.make_async_copy(XLb.at[pl.ds(j * 128, 128)],
                                  num_sp.at[pl.ds(0, 128)], ssemb).wait()
            pltpu.make_async_copy(wb.at[pl.ds(j * 128, 128)],
                                  den_sp.at[pl.ds(0, 128)], ssemb).wait()

    def _issue_scatter(g, XLb, wb, ssemb):
        for j in range(_RPC):
            pltpu.async_copy(XLb.at[pl.ds(j * 128, 128)],
                             num_sp.at[iall_dst.at[g * _RPC + j]], ssemb,
                             add=True)
            pltpu.async_copy(wb.at[pl.ds(j * 128, 128)],
                             den_sp.at[iall_dst.at[g * _RPC + j]], ssemb,
                             add=True)

    def _chunk_body(g, cur, nxt):
        XLb, XRb, wb, gsemb, ssemb = cur
        _wait_gather(XLb, XRb, gsemb)
        base = edge0 + g * C

        def _p1(i):
            eid = base + i
            m = XLb[i] + XRb[i]
            af = (1.0
                  + jnp.where(eid >= S1, 1.0, 0.0)
                  + jnp.where(eid >= S2, 1.0, 0.0))
            m = m + af * lew_v
            l = jnp.where(m > 0, m, 0.2 * m)
            lb = _bf16_round(l)
            cs = plsc.cumsum(lb * att_v)
            plsc.store_scatter(wb, [jnp.broadcast_to(i, (DG,))], cs,
                               mask=lane15)
        plsc.parallel_loop(0, C, 1, unroll=16)(_p1)

        def _p2(j):
            iv = j * DG
            ids = base + iv + lax.iota(jnp.int32, DG)
            valid = ((ids < E1)
                     | ((ids >= S1) & (ids < S1 + E1))
                     | ((ids >= S2) & (ids < S2 + E1)))
            s = wb[pl.ds(iv, DG)]
            wvec = jnp.where(valid, jnp.exp(s), 0.0)
            wb[pl.ds(iv, DG)] = wvec
            for k in range(DG):
                XLb[iv + k] = XLb[iv + k] * wvec[k]
        plsc.parallel_loop(0, C // DG, 1, unroll=4)(_p2)

        nXL, nXR, nw, ngsem, nssem = nxt

        @pl.when(g >= 1)
        def _():
            _drain_scatter(nXL, nw, nssem)

        @pl.when(g + 2 < CHUNKS)
        def _():
            _issue_gather(g + 2, nXL, nXR, ngsem)

        _issue_scatter(g, XLb, wb, ssemb)

    set0 = (XL0, XR0, w0, gsem0, ssem0)
    set1 = (XL1, XR1, w1, gsem1, ssem1)
    set2 = (XL2, XR2, w2, gsem2, ssem2)

    _issue_gather(0, XL0, XR0, gsem0)
    _issue_gather(1, XL1, XR1, gsem1)

    def _triple(p, _):
        g = 3 * p
        _chunk_body(g, set0, set2)
        _chunk_body(g + 1, set1, set0)
        _chunk_body(g + 2, set2, set1)
        return ()
    lax.fori_loop(0, (CHUNKS - 1) // 3, _triple, ())
    _chunk_body(CHUNKS - 1, set0, set2)
    _drain_scatter(XL0, w0, ssem0)

    plsc.subcore_barrier()

    out0 = cid * NP + row0
    pltpu.sync_copy(num_sp.at[pl.ds(row0, ROWS_PER_TILE)],
                    num_out.at[pl.ds(out0, ROWS_PER_TILE)])
    pltpu.sync_copy(den_sp.at[pl.ds(row0, ROWS_PER_TILE)],
                    den_out.at[pl.ds(out0, ROWS_PER_TILE)])


def _edge_aggregate(x_l, x_r, idx2, att, lew):
    mesh = plsc.VectorSubcoreMesh(core_axis_name="c", subcore_axis_name="s",
                                  num_cores=NC, num_subcores=NS)
    f = pl.kernel(
        _edge_kernel,
        out_type=[
            jax.ShapeDtypeStruct((NC * NP, DG), jnp.float32),
            jax.ShapeDtypeStruct((NC * NP,), jnp.float32),
        ],
        mesh=mesh,
        scratch_types=[
            pltpu.VMEM((T // 128, 128), jnp.int32),
            pltpu.VMEM((T // 128, 128), jnp.int32),
            pltpu.VMEM((C, DG), jnp.float32),
            pltpu.VMEM((C, DG), jnp.float32),
            pltpu.VMEM((C,), jnp.float32),
            pltpu.VMEM((C, DG), jnp.float32),
            pltpu.VMEM((C, DG), jnp.float32),
            pltpu.VMEM((C,), jnp.float32),
            pltpu.VMEM((C, DG), jnp.float32),
            pltpu.VMEM((C, DG), jnp.float32),
            pltpu.VMEM((C,), jnp.float32),
            pltpu.VMEM((DG,), jnp.float32),
            pltpu.VMEM((DG,), jnp.float32),
            pltpu.VMEM_SHARED((NP, DG), jnp.float32),
            pltpu.VMEM_SHARED((NP,), jnp.float32),
            pltpu.SemaphoreType.DMA,
            pltpu.SemaphoreType.DMA,
            pltpu.SemaphoreType.DMA,
            pltpu.SemaphoreType.DMA,
            pltpu.SemaphoreType.DMA,
            pltpu.SemaphoreType.DMA,
            pltpu.SemaphoreType.DMA,
        ],
        compiler_params=pltpu.CompilerParams(needs_layout_passes=False,
                                             use_tc_tiling_on_sc=False),
    )
    return f(x_l, x_r, idx2, att, lew)


def _post_body(n0, n1, d0, d1, gb, pw1, pb1, pw2, pb2, cw1, cb1, cw2, cb2,
               out):
    num = n0[0] + n1[0]
    den = d0[0] + d1[0]
    o = num / (den + 1e-16) + gb[...]
    h = jnp.tanh(o)
    h = _bdot(h, pw1[...]) + pb1[...]
    h = jnp.tanh(h)
    h = _bdot(h, pw2[...]) + pb2[...]
    h = _bdot(h, cw1[...]) + cb1[...]
    h = jnp.tanh(h)
    out[...] = _bdot(h, cw2[...]) + cb2[...]


def _post(num3, den3, gat_bias, proj_w1, proj_b1, proj_w2, proj_b2,
          cls_w1, cls_b1, cls_w2, cls_b2):
    blk = 2000
    dch = cls_w1.shape[1]
    full = lambda a: pl.BlockSpec(a.shape, lambda i: tuple(0 for _ in a.shape))
    return pl.pallas_call(
        _post_body,
        grid=(N // blk,),
        in_specs=[
            pl.BlockSpec((1, blk, DG), lambda i: (0, i, 0)),
            pl.BlockSpec((1, blk, DG), lambda i: (1, i, 0)),
            pl.BlockSpec((1, blk, 1), lambda i: (0, i, 0)),
            pl.BlockSpec((1, blk, 1), lambda i: (1, i, 0)),
            full(gat_bias), full(proj_w1), full(proj_b1), full(proj_w2),
            full(proj_b2), full(cls_w1), full(cls_b1), full(cls_w2),
            full(cls_b2),
        ],
        out_specs=pl.BlockSpec((blk, 2), lambda i: (i, 0)),
        out_shape=jax.ShapeDtypeStruct((N, 2), jnp.float32),
    )(num3, num3, den3, den3, gat_bias, proj_w1, proj_b1, proj_w2, proj_b2,
      cls_w1, cls_b1, cls_w2, cls_b2)


def kernel(x, edge_index_p, edge_index_s, edge_index_v, W_l, W_r, att,
           lin_edge_w, gat_bias, proj_w1, proj_b1, proj_w2, proj_b2,
           cls_w1, cls_b1, cls_w2, cls_b2):
    x_l, x_r = _project(x, W_l, W_r)

    z1 = jnp.zeros((2, S1 - E1), jnp.int32)
    z2 = jnp.zeros((2, E_PAD - S2 - E1), jnp.int32)
    idx2 = jnp.concatenate(
        [edge_index_p, z1, edge_index_s, z1, edge_index_v, z2],
        axis=1).reshape(2 * E_PAD // 128, 128)

    num_flat, den_flat = _edge_aggregate(
        x_l, x_r, idx2, att, lin_edge_w.reshape(DG))

    num3 = num_flat.reshape(NC, NP, DG)
    den3 = den_flat.reshape(NC, NP, 1)
    return _post(num3, den3, gat_bias, proj_w1, proj_b1, proj_w2, proj_b2,
                 cls_w1, cls_b1, cls_w2, cls_b2)


# speedup vs baseline: 1.0616x; 1.0616x over previous
import functools

import jax
import jax.numpy as jnp
from jax import lax
from jax.experimental import pallas as pl
from jax.experimental.pallas import tpu as pltpu
from jax.experimental.pallas import tpu_sc as plsc

N = 10000
D_IN = 128
DG = 16
E1 = 106667
E = 3 * E1
NC = 2
NS = 16
NW = NC * NS
C = 640
CHUNKS = 16
T = C * CHUNKS
E_PAD = NW * T
S1 = 109568
S2 = 219136
NP = 10240
ROWS_PER_TILE = NP // NS


def _bdot(a, b):
    return jnp.dot(a.astype(jnp.bfloat16), b.astype(jnp.bfloat16),
                   preferred_element_type=jnp.float32)


def _matmul_body(x_ref, wl_ref, wr_ref, xl_ref, xr_ref):
    xb = x_ref[...]
    xl_ref[...] = _bdot(xb, wl_ref[...])
    xr_ref[...] = _bdot(xb, wr_ref[...])


def _project(x, W_l, W_r):
    blk = 2000
    return pl.pallas_call(
        _matmul_body,
        grid=(N // blk,),
        in_specs=[
            pl.BlockSpec((blk, D_IN), lambda i: (i, 0)),
            pl.BlockSpec((D_IN, DG), lambda i: (0, 0)),
            pl.BlockSpec((D_IN, DG), lambda i: (0, 0)),
        ],
        out_specs=[
            pl.BlockSpec((blk, DG), lambda i: (i, 0)),
            pl.BlockSpec((blk, DG), lambda i: (i, 0)),
        ],
        out_shape=[
            jax.ShapeDtypeStruct((N, DG), jnp.float32),
            jax.ShapeDtypeStruct((N, DG), jnp.float32),
        ],
    )(x, W_l, W_r)


def _bf16_round(v):
    u = plsc.bitcast(v, jnp.uint32)
    u = ((u + jnp.uint32(0x7FFF) + ((u >> jnp.uint32(16)) & jnp.uint32(1)))
         & jnp.uint32(0xFFFF0000))
    return plsc.bitcast(u, jnp.float32)


_RPC = C // 128


def _edge_kernel(xl_hbm, xr_hbm, idx_hbm, att_hbm, lew_hbm,
                 num_out, den_out,
                 iall_src, iall_dst, XL0, XR0, w0, XL1, XR1, w1,
                 XL2, XR2, w2, attv, lewv,
                 num_sp, den_sp, isem, gsem0, gsem1, gsem2,
                 ssem0, ssem1, ssem2):
    cid = lax.axis_index("c")
    tid = lax.axis_index("s")
    wid = tid * NC + cid
    zero16 = jnp.zeros((DG,), jnp.float32)

    idx_row0 = wid * (T // 128)
    cpi = pltpu.async_copy(idx_hbm.at[pl.ds(idx_row0, T // 128)],
                           iall_src, isem)
    cpd = pltpu.async_copy(idx_hbm.at[pl.ds(E_PAD // 128 + idx_row0, T // 128)],
                           iall_dst, isem)

    def _z_rows(i):
        XL0[i] = zero16
    plsc.parallel_loop(0, ROWS_PER_TILE, 1, unroll=8)(_z_rows)

    def _z_w(i):
        w0[pl.ds(i * DG, DG)] = zero16
    plsc.parallel_loop(0, ROWS_PER_TILE // DG, 1, unroll=8)(_z_w)

    row0 = tid * ROWS_PER_TILE
    pltpu.sync_copy(XL0.at[pl.ds(0, ROWS_PER_TILE)],
                    num_sp.at[pl.ds(row0, ROWS_PER_TILE)])
    pltpu.sync_copy(w0.at[pl.ds(0, ROWS_PER_TILE)],
                    den_sp.at[pl.ds(row0, ROWS_PER_TILE)])

    pltpu.sync_copy(att_hbm, attv)
    pltpu.sync_copy(lew_hbm, lewv)
    att_v = _bf16_round(attv[...])
    lew_v = lewv[...]

    cpi.wait()
    cpd.wait()
    plsc.subcore_barrier()

    edge0 = wid * T
    lane15 = lax.iota(jnp.int32, DG) == DG - 1

    def _issue_gather(g, XLb, XRb, semb):
        for j in range(_RPC):
            pltpu.async_copy(xl_hbm.at[iall_src.at[g * _RPC + j]],
                             XLb.at[pl.ds(j * 128, 128)], semb)
            pltpu.async_copy(xr_hbm.at[iall_dst.at[g * _RPC + j]],
                             XRb.at[pl.ds(j * 128, 128)], semb)

    def _wait_gather(XLb, XRb, semb):
        for j in range(_RPC):
            pltpu.make_async_copy(xl_hbm.at[pl.ds(0, 128)],
                                  XLb.at[pl.ds(j * 128, 128)], semb).wait()
            pltpu.make_async_copy(xr_hbm.at[pl.ds(0, 128)],
                                  XRb.at[pl.ds(j * 128, 128)], semb).wait()

    def _drain_scatter(XLb, wb, ssemb):
        for j in range(_RPC):
            pltpu.make_async_copy(XLb.at[pl.ds(j * 128, 128)],
                                  num_sp.at[pl.ds(0, 128)], ssemb).wait()
            pltpu.make_async_copy(wb.at[pl.ds(j * 128, 128)],
                                  den_sp.at[pl.ds(0, 128)], ssemb).wait()

    def _issue_scatter(g, XLb, wb, ssemb):
        for j in range(_RPC):
            pltpu.async_copy(XLb.at[pl.ds(j * 128, 128)],
                             num_sp.at[iall_dst.at[g * _RPC + j]], ssemb,
                             add=True)
            pltpu.async_copy(wb.at[pl.ds(j * 128, 128)],
                             den_sp.at[iall_dst.at[g * _RPC + j]], ssemb,
                             add=True)

    def _chunk_body(g, cur, nxt):
        XLb, XRb, wb, gsemb, ssemb = cur
        _wait_gather(XLb, XRb, gsemb)
        base = edge0 + g * C

        def _p1(i):
            eid = base + i
            m = XLb[i] + XRb[i]
            af = (1.0
                  + jnp.where(eid >= S1, 1.0, 0.0)
                  + jnp.where(eid >= S2, 1.0, 0.0))
            m = m + af * lew_v
            l = jnp.where(m > 0, m, 0.2 * m)
            lb = _bf16_round(l)
            cs = plsc.cumsum(lb * att_v)
            plsc.store_scatter(wb, [jnp.broadcast_to(i, (DG,))], cs,
                               mask=lane15)
        plsc.parallel_loop(0, C, 1, unroll=4)(_p1)

        def _p2(j):
            iv = j * DG
            ids = base + iv + lax.iota(jnp.int32, DG)
            valid = ((ids < E1)
                     | ((ids >= S1) & (ids < S1 + E1))
                     | ((ids >= S2) & (ids < S2 + E1)))
            s = wb[pl.ds(iv, DG)]
            wvec = jnp.where(valid, jnp.exp(s), 0.0)
            wb[pl.ds(iv, DG)] = wvec
            for k in range(DG):
                XLb[iv + k] = XLb[iv + k] * wvec[k]
        plsc.parallel_loop(0, C // DG, 1, unroll=2)(_p2)

        nXL, nXR, nw, ngsem, nssem = nxt

        @pl.when(g >= 1)
        def _():
            _drain_scatter(nXL, nw, nssem)

        @pl.when(g + 2 < CHUNKS)
        def _():
            _issue_gather(g + 2, nXL, nXR, ngsem)

        _issue_scatter(g, XLb, wb, ssemb)

    set0 = (XL0, XR0, w0, gsem0, ssem0)
    set1 = (XL1, XR1, w1, gsem1, ssem1)
    set2 = (XL2, XR2, w2, gsem2, ssem2)

    _issue_gather(0, XL0, XR0, gsem0)
    _issue_gather(1, XL1, XR1, gsem1)

    def _triple(p, _):
        g = 3 * p
        _chunk_body(g, set0, set2)
        _chunk_body(g + 1, set1, set0)
        _chunk_body(g + 2, set2, set1)
        return ()
    lax.fori_loop(0, (CHUNKS - 1) // 3, _triple, ())
    _chunk_body(CHUNKS - 1, set0, set2)
    _drain_scatter(XL0, w0, ssem0)

    plsc.subcore_barrier()

    out0 = cid * NP + row0
    pltpu.sync_copy(num_sp.at[pl.ds(row0, ROWS_PER_TILE)],
                    num_out.at[pl.ds(out0, ROWS_PER_TILE)])
    pltpu.sync_copy(den_sp.at[pl.ds(row0, ROWS_PER_TILE)],
                    den_out.at[pl.ds(out0, ROWS_PER_TILE)])


def _edge_aggregate(x_l, x_r, idx2, att, lew):
    mesh = plsc.VectorSubcoreMesh(core_axis_name="c", subcore_axis_name="s",
                                  num_cores=NC, num_subcores=NS)
    f = pl.kernel(
        _edge_kernel,
        out_type=[
            jax.ShapeDtypeStruct((NC * NP, DG), jnp.float32),
            jax.ShapeDtypeStruct((NC * NP,), jnp.float32),
        ],
        mesh=mesh,
        scratch_types=[
            pltpu.VMEM((T // 128, 128), jnp.int32),
            pltpu.VMEM((T // 128, 128), jnp.int32),
            pltpu.VMEM((C, DG), jnp.float32),
            pltpu.VMEM((C, DG), jnp.float32),
            pltpu.VMEM((C,), jnp.float32),
            pltpu.VMEM((C, DG), jnp.float32),
            pltpu.VMEM((C, DG), jnp.float32),
            pltpu.VMEM((C,), jnp.float32),
            pltpu.VMEM((C, DG), jnp.float32),
            pltpu.VMEM((C, DG), jnp.float32),
            pltpu.VMEM((C,), jnp.float32),
            pltpu.VMEM((DG,), jnp.float32),
            pltpu.VMEM((DG,), jnp.float32),
            pltpu.VMEM_SHARED((NP, DG), jnp.float32),
            pltpu.VMEM_SHARED((NP,), jnp.float32),
            pltpu.SemaphoreType.DMA,
            pltpu.SemaphoreType.DMA,
            pltpu.SemaphoreType.DMA,
            pltpu.SemaphoreType.DMA,
            pltpu.SemaphoreType.DMA,
            pltpu.SemaphoreType.DMA,
            pltpu.SemaphoreType.DMA,
        ],
        compiler_params=pltpu.CompilerParams(needs_layout_passes=False,
                                             use_tc_tiling_on_sc=False),
    )
    return f(x_l, x_r, idx2, att, lew)


def _post_body(n0, n1, d0, d1, gb, pw1, pb1, pw2, pb2, cw1, cb1, cw2, cb2,
               out):
    num = n0[0] + n1[0]
    den = d0[0] + d1[0]
    o = num / (den + 1e-16) + gb[...]
    h = jnp.tanh(o)
    h = _bdot(h, pw1[...]) + pb1[...]
    h = jnp.tanh(h)
    h = _bdot(h, pw2[...]) + pb2[...]
    h = _bdot(h, cw1[...]) + cb1[...]
    h = jnp.tanh(h)
    out[...] = _bdot(h, cw2[...]) + cb2[...]


def _post(num3, den3, gat_bias, proj_w1, proj_b1, proj_w2, proj_b2,
          cls_w1, cls_b1, cls_w2, cls_b2):
    blk = 2000
    dch = cls_w1.shape[1]
    full = lambda a: pl.BlockSpec(a.shape, lambda i: tuple(0 for _ in a.shape))
    return pl.pallas_call(
        _post_body,
        grid=(N // blk,),
        in_specs=[
            pl.BlockSpec((1, blk, DG), lambda i: (0, i, 0)),
            pl.BlockSpec((1, blk, DG), lambda i: (1, i, 0)),
            pl.BlockSpec((1, blk, 1), lambda i: (0, i, 0)),
            pl.BlockSpec((1, blk, 1), lambda i: (1, i, 0)),
            full(gat_bias), full(proj_w1), full(proj_b1), full(proj_w2),
            full(proj_b2), full(cls_w1), full(cls_b1), full(cls_w2),
            full(cls_b2),
        ],
        out_specs=pl.BlockSpec((blk, 2), lambda i: (i, 0)),
        out_shape=jax.ShapeDtypeStruct((N, 2), jnp.float32),
    )(num3, num3, den3, den3, gat_bias, proj_w1, proj_b1, proj_w2, proj_b2,
      cls_w1, cls_b1, cls_w2, cls_b2)


def kernel(x, edge_index_p, edge_index_s, edge_index_v, W_l, W_r, att,
           lin_edge_w, gat_bias, proj_w1, proj_b1, proj_w2, proj_b2,
           cls_w1, cls_b1, cls_w2, cls_b2):
    x_l, x_r = _project(x, W_l, W_r)

    z1 = jnp.zeros((2, S1 - E1), jnp.int32)
    z2 = jnp.zeros((2, E_PAD - S2 - E1), jnp.int32)
    idx2 = jnp.concatenate(
        [edge_index_p, z1, edge_index_s, z1, edge_index_v, z2],
        axis=1).reshape(2 * E_PAD // 128, 128)

    num_flat, den_flat = _edge_aggregate(
        x_l, x_r, idx2, att, lin_edge_w.reshape(DG))

    num3 = num_flat.reshape(NC, NP, DG)
    den3 = den_flat.reshape(NC, NP, 1)
    return _post(num3, den3, gat_bias, proj_w1, proj_b1, proj_w2, proj_b2,
                 cls_w1, cls_b1, cls_w2, cls_b2)


# speedup vs baseline: 1.2116x; 1.1412x over previous
import functools

import jax
import jax.numpy as jnp
from jax import lax
from jax.experimental import pallas as pl
from jax.experimental.pallas import tpu as pltpu
from jax.experimental.pallas import tpu_sc as plsc

N = 10000
D_IN = 128
DG = 16
E1 = 106667
E = 3 * E1
NC = 2
NS = 16
NW = NC * NS
C = 640
CHUNKS = 16
T = C * CHUNKS
E_PAD = NW * T
S1 = 109568
S2 = 219136
NP = 10240
ROWS_PER_TILE = NP // NS


def _bdot(a, b):
    return jnp.dot(a.astype(jnp.bfloat16), b.astype(jnp.bfloat16),
                   preferred_element_type=jnp.float32)


def _matmul_body(x_ref, wl_ref, wr_ref, eip_ref, eis_ref, eiv_ref,
                 xl_ref, xr_ref, idx_ref):
    xb = x_ref[...]
    xl_ref[...] = _bdot(xb, wl_ref[...])
    xr_ref[...] = _bdot(xb, wr_ref[...])

    @pl.when(pl.program_id(0) == 0)
    def _():
        idx_ref[...] = jnp.zeros((2 * E_PAD,), jnp.int32)
        for r, half in ((0, 0), (1, E_PAD)):
            idx_ref[pl.ds(half, E1)] = eip_ref[r]
            idx_ref[pl.ds(half + S1, E1)] = eis_ref[r]
            idx_ref[pl.ds(half + S2, E1)] = eiv_ref[r]


def _project(x, W_l, W_r, eip, eis, eiv):
    blk = 2000
    full = lambda a: pl.BlockSpec(a.shape, lambda i: tuple(0 for _ in a.shape))
    return pl.pallas_call(
        _matmul_body,
        grid=(N // blk,),
        in_specs=[
            pl.BlockSpec((blk, D_IN), lambda i: (i, 0)),
            pl.BlockSpec((D_IN, DG), lambda i: (0, 0)),
            pl.BlockSpec((D_IN, DG), lambda i: (0, 0)),
            full(eip), full(eis), full(eiv),
        ],
        out_specs=[
            pl.BlockSpec((blk, DG), lambda i: (i, 0)),
            pl.BlockSpec((blk, DG), lambda i: (i, 0)),
            pl.BlockSpec((2 * E_PAD,), lambda i: (0,)),
        ],
        out_shape=[
            jax.ShapeDtypeStruct((N, DG), jnp.float32),
            jax.ShapeDtypeStruct((N, DG), jnp.float32),
            jax.ShapeDtypeStruct((2 * E_PAD,), jnp.int32),
        ],
    )(x, W_l, W_r, eip, eis, eiv)


def _bf16_round(v):
    u = plsc.bitcast(v, jnp.uint32)
    u = ((u + jnp.uint32(0x7FFF) + ((u >> jnp.uint32(16)) & jnp.uint32(1)))
         & jnp.uint32(0xFFFF0000))
    return plsc.bitcast(u, jnp.float32)


_RPC = C // 128


def _edge_kernel(xl_hbm, xr_hbm, idx_hbm, att_hbm, lew_hbm,
                 num_out, den_out,
                 iall_src, iall_dst, XL0, XR0, w0, XL1, XR1, w1,
                 XL2, XR2, w2, attv, lewv,
                 num_sp, den_sp, isem, gsem0, gsem1, gsem2,
                 ssem0, ssem1, ssem2):
    cid = lax.axis_index("c")
    tid = lax.axis_index("s")
    wid = tid * NC + cid
    zero16 = jnp.zeros((DG,), jnp.float32)

    idx_row0 = wid * (T // 128)
    cpi = pltpu.async_copy(idx_hbm.at[pl.ds(idx_row0, T // 128)],
                           iall_src, isem)
    cpd = pltpu.async_copy(idx_hbm.at[pl.ds(E_PAD // 128 + idx_row0, T // 128)],
                           iall_dst, isem)

    def _z_rows(i):
        XL0[i] = zero16
    plsc.parallel_loop(0, ROWS_PER_TILE, 1, unroll=8)(_z_rows)

    def _z_w(i):
        w0[pl.ds(i * DG, DG)] = zero16
    plsc.parallel_loop(0, ROWS_PER_TILE // DG, 1, unroll=8)(_z_w)

    row0 = tid * ROWS_PER_TILE
    pltpu.sync_copy(XL0.at[pl.ds(0, ROWS_PER_TILE)],
                    num_sp.at[pl.ds(row0, ROWS_PER_TILE)])
    pltpu.sync_copy(w0.at[pl.ds(0, ROWS_PER_TILE)],
                    den_sp.at[pl.ds(row0, ROWS_PER_TILE)])

    pltpu.sync_copy(att_hbm, attv)
    pltpu.sync_copy(lew_hbm, lewv)
    att_v = _bf16_round(attv[...])
    lew_v = lewv[...]

    cpi.wait()
    cpd.wait()
    plsc.subcore_barrier()

    edge0 = wid * T
    lane15 = lax.iota(jnp.int32, DG) == DG - 1

    def _issue_gather(g, XLb, XRb, semb):
        for j in range(_RPC):
            pltpu.async_copy(xl_hbm.at[iall_src.at[g * _RPC + j]],
                             XLb.at[pl.ds(j * 128, 128)], semb)
            pltpu.async_copy(xr_hbm.at[iall_dst.at[g * _RPC + j]],
                             XRb.at[pl.ds(j * 128, 128)], semb)

    def _wait_gather(XLb, XRb, semb):
        for j in range(_RPC):
            pltpu.make_async_copy(xl_hbm.at[pl.ds(0, 128)],
                                  XLb.at[pl.ds(j * 128, 128)], semb).wait()
            pltpu.make_async_copy(xr_hbm.at[pl.ds(0, 128)],
                                  XRb.at[pl.ds(j * 128, 128)], semb).wait()

    def _drain_scatter(XLb, wb, ssemb):
        for j in range(_RPC):
            pltpu.make_async_copy(XLb.at[pl.ds(j * 128, 128)],
                                  num_sp.at[pl.ds(0, 128)], ssemb).wait()
            pltpu.make_async_copy(wb.at[pl.ds(j * 128, 128)],
                                  den_sp.at[pl.ds(0, 128)], ssemb).wait()

    def _issue_scatter(g, XLb, wb, ssemb):
        for j in range(_RPC):
            pltpu.async_copy(XLb.at[pl.ds(j * 128, 128)],
                             num_sp.at[iall_dst.at[g * _RPC + j]], ssemb,
                             add=True)
            pltpu.async_copy(wb.at[pl.ds(j * 128, 128)],
                             den_sp.at[iall_dst.at[g * _RPC + j]], ssemb,
                             add=True)

    def _chunk_body(g, cur, nxt):
        XLb, XRb, wb, gsemb, ssemb = cur
        _wait_gather(XLb, XRb, gsemb)
        base = edge0 + g * C

        def _p1(i):
            eid = base + i
            m = XLb[i] + XRb[i]
            af = (1.0
                  + jnp.where(eid >= S1, 1.0, 0.0)
                  + jnp.where(eid >= S2, 1.0, 0.0))
            m = m + af * lew_v
            l = jnp.where(m > 0, m, 0.2 * m)
            lb = _bf16_round(l)
            cs = plsc.cumsum(lb * att_v)
            plsc.store_scatter(wb, [jnp.broadcast_to(i, (DG,))], cs,
                               mask=lane15)
        plsc.parallel_loop(0, C, 1, unroll=4)(_p1)

        def _p2(j):
            iv = j * DG
            ids = base + iv + lax.iota(jnp.int32, DG)
            valid = ((ids < E1)
                     | ((ids >= S1) & (ids < S1 + E1))
                     | ((ids >= S2) & (ids < S2 + E1)))
            s = wb[pl.ds(iv, DG)]
            wvec = jnp.where(valid, jnp.exp(s), 0.0)
            wb[pl.ds(iv, DG)] = wvec
            for k in range(DG):
                XLb[iv + k] = XLb[iv + k] * wvec[k]
        plsc.parallel_loop(0, C // DG, 1, unroll=2)(_p2)

        nXL, nXR, nw, ngsem, nssem = nxt

        @pl.when(g >= 1)
        def _():
            _drain_scatter(nXL, nw, nssem)

        @pl.when(g + 2 < CHUNKS)
        def _():
            _issue_gather(g + 2, nXL, nXR, ngsem)

        _issue_scatter(g, XLb, wb, ssemb)

    set0 = (XL0, XR0, w0, gsem0, ssem0)
    set1 = (XL1, XR1, w1, gsem1, ssem1)
    set2 = (XL2, XR2, w2, gsem2, ssem2)

    _issue_gather(0, XL0, XR0, gsem0)
    _issue_gather(1, XL1, XR1, gsem1)

    def _triple(p, _):
        g = 3 * p
        _chunk_body(g, set0, set2)
        _chunk_body(g + 1, set1, set0)
        _chunk_body(g + 2, set2, set1)
        return ()
    lax.fori_loop(0, (CHUNKS - 1) // 3, _triple, ())
    _chunk_body(CHUNKS - 1, set0, set2)
    _drain_scatter(XL0, w0, ssem0)

    plsc.subcore_barrier()

    out0 = cid * NP + row0
    pltpu.sync_copy(num_sp.at[pl.ds(row0, ROWS_PER_TILE)],
                    num_out.at[pl.ds(out0, ROWS_PER_TILE)])
    pltpu.sync_copy(den_sp.at[pl.ds(row0, ROWS_PER_TILE)],
                    den_out.at[pl.ds(out0, ROWS_PER_TILE)])


def _edge_aggregate(x_l, x_r, idx2, att, lew):
    mesh = plsc.VectorSubcoreMesh(core_axis_name="c", subcore_axis_name="s",
                                  num_cores=NC, num_subcores=NS)
    f = pl.kernel(
        _edge_kernel,
        out_type=[
            jax.ShapeDtypeStruct((NC * NP, DG), jnp.float32),
            jax.ShapeDtypeStruct((NC * NP,), jnp.float32),
        ],
        mesh=mesh,
        scratch_types=[
            pltpu.VMEM((T // 128, 128), jnp.int32),
            pltpu.VMEM((T // 128, 128), jnp.int32),
            pltpu.VMEM((C, DG), jnp.float32),
            pltpu.VMEM((C, DG), jnp.float32),
            pltpu.VMEM((C,), jnp.float32),
            pltpu.VMEM((C, DG), jnp.float32),
            pltpu.VMEM((C, DG), jnp.float32),
            pltpu.VMEM((C,), jnp.float32),
            pltpu.VMEM((C, DG), jnp.float32),
            pltpu.VMEM((C, DG), jnp.float32),
            pltpu.VMEM((C,), jnp.float32),
            pltpu.VMEM((DG,), jnp.float32),
            pltpu.VMEM((DG,), jnp.float32),
            pltpu.VMEM_SHARED((NP, DG), jnp.float32),
            pltpu.VMEM_SHARED((NP,), jnp.float32),
            pltpu.SemaphoreType.DMA,
            pltpu.SemaphoreType.DMA,
            pltpu.SemaphoreType.DMA,
            pltpu.SemaphoreType.DMA,
            pltpu.SemaphoreType.DMA,
            pltpu.SemaphoreType.DMA,
            pltpu.SemaphoreType.DMA,
        ],
        compiler_params=pltpu.CompilerParams(needs_layout_passes=False,
                                             use_tc_tiling_on_sc=False),
    )
    return f(x_l, x_r, idx2, att, lew)


def _post_body(n0, n1, d0, d1, gb, pw1, pb1, pw2, pb2, cw1, cb1, cw2, cb2,
               out):
    num = n0[0] + n1[0]
    den = d0[0] + d1[0]
    o = num / (den + 1e-16) + gb[...]
    h = jnp.tanh(o)
    h = _bdot(h, pw1[...]) + pb1[...]
    h = jnp.tanh(h)
    h = _bdot(h, pw2[...]) + pb2[...]
    h = _bdot(h, cw1[...]) + cb1[...]
    h = jnp.tanh(h)
    out[...] = _bdot(h, cw2[...]) + cb2[...]


def _post(num3, den3, gat_bias, proj_w1, proj_b1, proj_w2, proj_b2,
          cls_w1, cls_b1, cls_w2, cls_b2):
    blk = 2000
    dch = cls_w1.shape[1]
    full = lambda a: pl.BlockSpec(a.shape, lambda i: tuple(0 for _ in a.shape))
    return pl.pallas_call(
        _post_body,
        grid=(N // blk,),
        in_specs=[
            pl.BlockSpec((1, blk, DG), lambda i: (0, i, 0)),
            pl.BlockSpec((1, blk, DG), lambda i: (1, i, 0)),
            pl.BlockSpec((1, blk, 1), lambda i: (0, i, 0)),
            pl.BlockSpec((1, blk, 1), lambda i: (1, i, 0)),
            full(gat_bias), full(proj_w1), full(proj_b1), full(proj_w2),
            full(proj_b2), full(cls_w1), full(cls_b1), full(cls_w2),
            full(cls_b2),
        ],
        out_specs=pl.BlockSpec((blk, 2), lambda i: (i, 0)),
        out_shape=jax.ShapeDtypeStruct((N, 2), jnp.float32),
    )(num3, num3, den3, den3, gat_bias, proj_w1, proj_b1, proj_w2, proj_b2,
      cls_w1, cls_b1, cls_w2, cls_b2)


def kernel(x, edge_index_p, edge_index_s, edge_index_v, W_l, W_r, att,
           lin_edge_w, gat_bias, proj_w1, proj_b1, proj_w2, proj_b2,
           cls_w1, cls_b1, cls_w2, cls_b2):
    x_l, x_r, idx_flat = _project(x, W_l, W_r, edge_index_p,
                                   edge_index_s, edge_index_v)
    idx2 = idx_flat.reshape(2 * E_PAD // 128, 128)

    num_flat, den_flat = _edge_aggregate(
        x_l, x_r, idx2, att, lin_edge_w.reshape(DG))

    num3 = num_flat.reshape(NC, NP, DG)
    den3 = den_flat.reshape(NC, NP, 1)
    return _post(num3, den3, gat_bias, proj_w1, proj_b1, proj_w2, proj_b2,
                 cls_w1, cls_b1, cls_w2, cls_b2)
